# baseline (speedup 1.0000x reference)
import functools

import jax
import jax.numpy as jnp
from jax import lax
from jax.experimental import pallas as pl
from jax.experimental.pallas import tpu as pltpu
from jax.experimental.pallas import tpu_sc as plsc

K = 100000
D = 256
L = 16
NC = 2
NS = 16
NW = NC * NS
CHUNK = 160
NCHUNK = K // CHUNK
NBUF = 3
NG = CHUNK // L
NVEC = D // L
RI = 10

_NEG = -3.0e38


def _splat_f(x):
    return jnp.full((L,), x, dtype=jnp.float32)


def _splat_i(x):
    return jnp.full((L,), x, dtype=jnp.int32)


def _rsqrt16(x, iters=3):
    i = lax.bitcast_convert_type(x, jnp.int32)
    i = _splat_i(0x5F3759DF) - lax.shift_right_arithmetic(i, _splat_i(1))
    y = lax.bitcast_convert_type(i, jnp.float32)
    half_x = _splat_f(0.5) * x
    for _ in range(iters):
        y = y * (_splat_f(1.5) - half_x * y * y)
    return y


def _router_body(q_hbm, keys_hbm, out_s_hbm, out_i_hbm,
                 buf, qv, dots, n2s, bs_ref, bi_ref,
                 sem0, sem1, sem2):
    sems = (sem0, sem1, sem2)
    wid = lax.axis_index("s") * NC + lax.axis_index("c")
    n_w = jnp.where(wid < NCHUNK % NW, NCHUNK // NW + 1, NCHUNK // NW)

    pltpu.sync_copy(q_hbm, qv)
    qs = [qv[pl.ds(L * j, L)] for j in range(NVEC)]
    aq = qs[0] * qs[0]
    for j in range(1, NVEC):
        aq = aq + qs[j] * qs[j]
    q2 = jnp.full((L,), jnp.sum(aq), dtype=jnp.float32)
    qn = q2 * _rsqrt16(q2)
    inv_qd = _splat_f(1.0) / (qn + _splat_f(1e-8))

    bs_ref[...] = _splat_f(_NEG)
    bi_ref[...] = _splat_i(0)

    H = CHUNK // 2

    def _start(t, b):
        base = (wid + t * NW) * CHUNK
        pltpu.async_copy(
            keys_hbm.at[pl.ds(base, H)],
            buf.at[b, pl.ds(0, H)],
            sems[b])
        pltpu.async_copy(
            keys_hbm.at[pl.ds(base + H, H)],
            buf.at[b, pl.ds(H, H)],
            sems[b])

    def _wait(b):
        for h in range(2):
            pltpu.make_async_copy(
                keys_hbm.at[pl.ds(0, H)],
                buf.at[b, pl.ds(h * H, H)],
                sems[b]).wait()

    lane = lax.iota(jnp.int32, L)
    last_lane = lane == _splat_i(L - 1)

    def _phase1(b, nrows):
        def row_body(i, carry):
            r0 = i * RI
            ads = [None] * RI
            ans = [None] * RI
            for u in range(NVEC + RI):
                for dr in range(RI):
                    j = u - dr
                    if j < 0 or j >= NVEC:
                        continue
                    v = buf[b, r0 + dr, pl.ds(L * j, L)]
                    if j == 0:
                        ads[dr] = v * qs[0]
                        ans[dr] = v * v
                    else:
                        ads[dr] = ads[dr] + v * qs[j]
                        ans[dr] = ans[dr] + v * v
                    if j == NVEC - 1:
                        cd = plsc.cumsum(ads[dr])
                        cn = plsc.cumsum(ans[dr])
                        rv = jnp.full((L,), r0 + dr, dtype=jnp.int32)
                        plsc.store_scatter(dots, [rv], cd, mask=last_lane)
                        plsc.store_scatter(n2s, [rv], cn, mask=last_lane)
            return carry

        lax.fori_loop(0, nrows // RI, row_body, 0)

    def _phase2(base, ngroups):
        for g in range(ngroups):
            dv = dots[pl.ds(g * L, L)]
            nv = n2s[pl.ds(g * L, L)]
            s = nv * _rsqrt16(nv, iters=2)
            sim = (dv * inv_qd) / (s + _splat_f(1e-8))
            idxv = jnp.full((L,), base + g * L, dtype=jnp.int32) + lane
            bs = bs_ref[...]
            upd = sim > bs
            bs_ref[...] = jnp.where(upd, sim, bs)
            bi_ref[...] = jnp.where(upd, idxv, bi_ref[...])

    def _score_chunk(t, b):
        _phase1(b, CHUNK)
        _phase2((wid + t * NW) * CHUNK, NG)

    for b in range(NBUF):
        _start(b, b)

    def outer(i, carry):
        for b in range(NBUF):
            t = i * NBUF + b

            @pl.when(t < n_w)
            def _():
                _wait(b)
                _score_chunk(t, b)

                @pl.when(t + NBUF < n_w)
                def _():
                    _start(t + NBUF, b)

        return carry

    n_outer = -(-(NCHUNK // NW + 1) // NBUF)
    lax.fori_loop(0, n_outer, outer, 0)

    bs = bs_ref[...]
    m = jnp.full((L,), jnp.max(bs), dtype=jnp.float32)
    cand = jnp.where(bs == m, bi_ref[...], _splat_i(2147483647))
    bidx = jnp.min(cand)
    bs_ref[...] = m
    bi_ref[...] = jnp.full((L,), bidx, dtype=jnp.int32)
    pltpu.sync_copy(bs_ref, out_s_hbm.at[pl.ds(wid * L, L)])
    pltpu.sync_copy(bi_ref, out_i_hbm.at[pl.ds(wid * L, L)])


_router = functools.partial(
    pl.kernel,
    mesh=plsc.VectorSubcoreMesh(core_axis_name="c", subcore_axis_name="s"),
    compiler_params=pltpu.CompilerParams(needs_layout_passes=False),
    out_type=[
        jax.ShapeDtypeStruct((NW * L,), jnp.float32),
        jax.ShapeDtypeStruct((NW * L,), jnp.int32),
    ],
    scratch_types=[
        pltpu.VMEM((NBUF, CHUNK, D), jnp.float32),
        pltpu.VMEM((D,), jnp.float32),
        pltpu.VMEM((CHUNK,), jnp.float32),
        pltpu.VMEM((CHUNK,), jnp.float32),
        pltpu.VMEM((L,), jnp.float32),
        pltpu.VMEM((L,), jnp.int32),
        pltpu.SemaphoreType.DMA,
        pltpu.SemaphoreType.DMA,
        pltpu.SemaphoreType.DMA,
    ],
)(_router_body)


def kernel(query_embedding, keys):
    out_s, out_i = _router(query_embedding, keys)
    m = jnp.max(out_s)
    bi = jnp.min(jnp.where(out_s == m, out_i, jnp.int32(2147483647)))
    return bi, m


# speedup vs baseline: 1.2856x; 1.2856x over previous
import functools

import jax
import jax.numpy as jnp
from jax import lax
from jax.experimental import pallas as pl
from jax.experimental.pallas import tpu as pltpu
from jax.experimental.pallas import tpu_sc as plsc

K = 100000
D = 256
L = 16
NC = 2
NS = 16
NW = NC * NS
CHUNK = 160
NCHUNK = K // CHUNK
NBUF = 3
NG = CHUNK // L
NVEC = D // L
RI = 10

_NEG = -3.0e38


def _splat_f(x):
    return jnp.full((L,), x, dtype=jnp.float32)


def _splat_i(x):
    return jnp.full((L,), x, dtype=jnp.int32)


def _rsqrt16(x, iters=3):
    i = lax.bitcast_convert_type(x, jnp.int32)
    i = _splat_i(0x5F3759DF) - lax.shift_right_arithmetic(i, _splat_i(1))
    y = lax.bitcast_convert_type(i, jnp.float32)
    half_x = _splat_f(0.5) * x
    for _ in range(iters):
        y = y * (_splat_f(1.5) - half_x * y * y)
    return y


def _router_body(q_hbm, keys_hbm, out_s_hbm, out_i_hbm,
                 buf, qv, dots, n2s, bs_ref, bi_ref,
                 sem0, sem1, sem2):
    sems = (sem0, sem1, sem2)
    wid = lax.axis_index("s") * NC + lax.axis_index("c")
    n_w = jnp.where(wid < NCHUNK % NW, NCHUNK // NW + 1, NCHUNK // NW)

    pltpu.sync_copy(q_hbm, qv)
    qs = [qv[pl.ds(L * j, L)] for j in range(NVEC)]
    aq = qs[0] * qs[0]
    for j in range(1, NVEC):
        aq = aq + qs[j] * qs[j]
    q2 = jnp.full((L,), jnp.sum(aq), dtype=jnp.float32)
    qn = q2 * _rsqrt16(q2)
    inv_qd = _splat_f(1.0) / (qn + _splat_f(1e-8))

    bs_ref[...] = _splat_f(_NEG)
    bi_ref[...] = _splat_i(0)

    H = CHUNK // 2

    def _start(t, b):
        base = (wid + t * NW) * CHUNK
        pltpu.async_copy(
            keys_hbm.at[pl.ds(base, H)],
            buf.at[b, pl.ds(0, H)],
            sems[b])
        pltpu.async_copy(
            keys_hbm.at[pl.ds(base + H, H)],
            buf.at[b, pl.ds(H, H)],
            sems[b])

    def _wait(b):
        for h in range(2):
            pltpu.make_async_copy(
                keys_hbm.at[pl.ds(0, H)],
                buf.at[b, pl.ds(h * H, H)],
                sems[b]).wait()

    lane = lax.iota(jnp.int32, L)
    last_lane = lane == _splat_i(L - 1)

    def _phase1(b, nrows):
        def row_body(i, carry):
            r0 = i * RI
            ads = [None] * RI
            ans = [None] * RI
            for j in range(NVEC):
                for dr in range(RI):
                    v = buf[b, r0 + dr, pl.ds(L * j, L)]
                    if j == 0:
                        ads[dr] = v * qs[0]
                        ans[dr] = v * v
                    else:
                        ads[dr] = ads[dr] + v * qs[j]
                        ans[dr] = ans[dr] + v * v
            for dr in range(RI):
                cd = plsc.cumsum(ads[dr])
                cn = plsc.cumsum(ans[dr])
                rv = jnp.full((L,), r0 + dr, dtype=jnp.int32)
                plsc.store_scatter(dots, [rv], cd, mask=last_lane)
                plsc.store_scatter(n2s, [rv], cn, mask=last_lane)
            return carry

        lax.fori_loop(0, nrows // RI, row_body, 0)

    def _phase2(base, ngroups):
        for g in range(ngroups):
            dv = dots[pl.ds(g * L, L)]
            nv = n2s[pl.ds(g * L, L)]
            s = nv * _rsqrt16(nv, iters=2)
            sim = (dv * inv_qd) / (s + _splat_f(1e-8))
            idxv = jnp.full((L,), base + g * L, dtype=jnp.int32) + lane
            bs = bs_ref[...]
            upd = sim > bs
            bs_ref[...] = jnp.where(upd, sim, bs)
            bi_ref[...] = jnp.where(upd, idxv, bi_ref[...])

    def _score_chunk(t, b):
        _phase1(b, CHUNK)
        _phase2((wid + t * NW) * CHUNK, NG)

    for b in range(NBUF):
        _start(b, b)

    def outer(i, carry):
        for b in range(NBUF):
            t = i * NBUF + b

            @pl.when(t < n_w)
            def _():
                _wait(b)
                _score_chunk(t, b)

                @pl.when(t + NBUF < n_w)
                def _():
                    _start(t + NBUF, b)

        return carry

    n_outer = -(-(NCHUNK // NW + 1) // NBUF)
    lax.fori_loop(0, n_outer, outer, 0)

    bs = bs_ref[...]
    m = jnp.full((L,), jnp.max(bs), dtype=jnp.float32)
    cand = jnp.where(bs == m, bi_ref[...], _splat_i(2147483647))
    bidx = jnp.min(cand)
    bs_ref[...] = m
    bi_ref[...] = jnp.full((L,), bidx, dtype=jnp.int32)
    pltpu.sync_copy(bs_ref, out_s_hbm.at[pl.ds(wid * L, L)])
    pltpu.sync_copy(bi_ref, out_i_hbm.at[pl.ds(wid * L, L)])


_router = functools.partial(
    pl.kernel,
    mesh=plsc.VectorSubcoreMesh(core_axis_name="c", subcore_axis_name="s"),
    compiler_params=pltpu.CompilerParams(needs_layout_passes=False),
    out_type=[
        jax.ShapeDtypeStruct((NW * L,), jnp.float32),
        jax.ShapeDtypeStruct((NW * L,), jnp.int32),
    ],
    scratch_types=[
        pltpu.VMEM((NBUF, CHUNK, D), jnp.float32),
        pltpu.VMEM((D,), jnp.float32),
        pltpu.VMEM((CHUNK,), jnp.float32),
        pltpu.VMEM((CHUNK,), jnp.float32),
        pltpu.VMEM((L,), jnp.float32),
        pltpu.VMEM((L,), jnp.int32),
        pltpu.SemaphoreType.DMA,
        pltpu.SemaphoreType.DMA,
        pltpu.SemaphoreType.DMA,
    ],
)(_router_body)


def kernel(query_embedding, keys):
    out_s, out_i = _router(query_embedding, keys)
    m = jnp.max(out_s)
    bi = jnp.min(jnp.where(out_s == m, out_i, jnp.int32(2147483647)))
    return bi, m
